# baseline (speedup 1.0000x reference)
import jax
import jax.numpy as jnp
from jax.experimental import pallas as pl

N = 10000
D = 128
K = 10
NEG_SLOPE = 0.01


def _axpy_kernel(th_ref, h_ref, t_ref, o_ref):
    o_ref[...] = th_ref[...] + h_ref[...] * t_ref[0]


def _axpy(th, h, t):
    return pl.pallas_call(
        _axpy_kernel,
        out_shape=jax.ShapeDtypeStruct(th.shape, th.dtype),
    )(th, h, t.reshape(1))


def kernel(in_feat, edge_index, temp, w_attn):
    src = edge_index[0]
    dst = edge_index[1]
    order = jnp.argsort(dst)
    src = src[order]
    dst = dst[order]
    w_src = w_attn[:D]
    w_dst = w_attn[D:]
    h = in_feat
    th = in_feat * temp[0]
    for k in range(K):
        e = h[src] @ w_src + h[dst] @ w_dst
        e = jnp.where(e > 0, e, NEG_SLOPE * e)
        m = jax.ops.segment_max(e, dst, num_segments=N)
        ex = jnp.exp(e - m[dst])
        s = jax.ops.segment_sum(ex, dst, num_segments=N)
        a = ex / s[dst]
        h = jax.ops.segment_sum(h[src] * a[:, None], dst, num_segments=N)
        th = _axpy(th, h, temp[k + 1])
    return th


# speedup vs baseline: 8.3300x; 8.3300x over previous
import functools

import jax
import jax.numpy as jnp
from jax import lax
from jax.experimental import pallas as pl
from jax.experimental.pallas import tpu as pltpu
from jax.experimental.pallas import tpu_sc as plsc

N = 10000
D = 128
E = 320000
K = 10
NEG_SLOPE = 0.01

NC, NS, L = 2, 16, 16
T = NC * NS
NPT = 320
NPAD = NPT * T
B = 128
EPAD = E + B
U = D // L

_mesh = plsc.VectorSubcoreMesh(
    core_axis_name="c", subcore_axis_name="s", num_cores=NC, num_subcores=NS
)

_f32 = jnp.float32
_i32 = jnp.int32


def _wid():
    return lax.axis_index("c") * NS + lax.axis_index("s")


def _iota():
    return lax.iota(_i32, L)


_GDN = lax.GatherDimensionNumbers(
    offset_dims=(), collapsed_slice_dims=(0,), start_index_map=(0,)
)


def _rot(v, sh):
    idx = lax.iota(_i32, L)
    idx = (idx + sh) & (L - 1)
    return lax.gather(v, idx[:, None], _GDN, slice_sizes=(1,),
                      mode=lax.GatherScatterMode.PROMISE_IN_BOUNDS)


def _hsum(v):
    for sh in (8, 4, 2, 1):
        v = v + _rot(v, sh)
    return v


@functools.partial(
    pl.kernel,
    out_type=(
        jax.ShapeDtypeStruct((NPAD,), _f32),
        jax.ShapeDtypeStruct((NPAD,), _f32),
        jax.ShapeDtypeStruct((NPAD, D), _f32),
    ),
    mesh=_mesh,
    compiler_params=pltpu.CompilerParams(needs_layout_passes=False),
    scratch_types=[
        pltpu.VMEM((64, D), _f32),
        pltpu.VMEM((NPT,), _f32),
        pltpu.VMEM((NPT,), _f32),
        pltpu.VMEM((2 * D,), _f32),
        pltpu.VMEM((16,), _f32),
    ],
)
def _init_kernel(h_hbm, w_hbm, t_hbm, p_hbm, q_hbm, th_hbm, rbuf, pb, qb, wbuf, tbuf):
    base = pl.multiple_of(_wid() * NPT, 8)
    pltpu.sync_copy(w_hbm, wbuf)
    pltpu.sync_copy(t_hbm, tbuf)
    t0 = tbuf[pl.ds(0, L)][0]
    for cc in range(NPT // 64):
        rbase = cc * 64
        pltpu.sync_copy(h_hbm.at[pl.ds(base + rbase, 64), :], rbuf)

        def grp_body(g, c, rbase=rbase):
            pvec = jnp.zeros((L,), _f32)
            qvec = jnp.zeros((L,), _f32)
            for j in range(L):
                i = g * L + j
                dp = jnp.zeros((L,), _f32)
                dq = jnp.zeros((L,), _f32)
                for u in range(U):
                    sl = pl.ds(u * L, L)
                    hv = rbuf[i, sl]
                    dp = dp + hv * wbuf[sl]
                    dq = dq + hv * wbuf[pl.ds(D + u * L, L)]
                    rbuf[i, sl] = hv * t0
                ej = (_iota() == j)
                pvec = jnp.where(ej, _hsum(dp), pvec)
                qvec = jnp.where(ej, _hsum(dq), qvec)
            pb[pl.ds(rbase + g * L, L)] = pvec
            qb[pl.ds(rbase + g * L, L)] = qvec
            return c

        lax.fori_loop(0, 64 // L, grp_body, 0)
        pltpu.sync_copy(rbuf, th_hbm.at[pl.ds(base + rbase, 64), :])
    pltpu.sync_copy(pb, p_hbm.at[pl.ds(base, NPT)])
    pltpu.sync_copy(qb, q_hbm.at[pl.ds(base, NPT)])


@functools.partial(
    pl.kernel,
    out_type=(
        jax.ShapeDtypeStruct((NPAD, D), _f32),
        jax.ShapeDtypeStruct((NPAD, D), _f32),
        jax.ShapeDtypeStruct((NPAD,), _f32),
        jax.ShapeDtypeStruct((NPAD,), _f32),
    ),
    mesh=_mesh,
    compiler_params=pltpu.CompilerParams(needs_layout_passes=False),
    scratch_types=[
        pltpu.VMEM((NPAD,), _f32),
        pltpu.VMEM((NPAD,), _f32),
        pltpu.VMEM((NPT, D), _f32),
        pltpu.VMEM((NPT + L,), _f32),
        pltpu.VMEM((NPT + L,), _f32),
        pltpu.VMEM((B, D), _f32),
        pltpu.VMEM((B,), _i32),
        pltpu.VMEM((B,), _i32),
        pltpu.VMEM((B,), _f32),
        pltpu.VMEM((64, D), _f32),
        pltpu.VMEM((2 * D,), _f32),
        pltpu.VMEM((16,), _f32),
        pltpu.VMEM((48,), _i32),
        pltpu.VMEM((NPT,), _f32),
        pltpu.VMEM((NPT,), _f32),
        pltpu.SemaphoreType.DMA,
    ],
)
def _step_kernel(h_hbm, th_hbm, p_hbm, q_hbm, src_hbm, dst_hbm, es_hbm, w_hbm,
                 t_hbm, h_out, th_out, p_out, q_out,
                 pbuf, qbuf, acc, svec, rsb, rows, sidx, didx, exb, thbuf,
                 wbuf, tbuf, esb, pb2, qb2, sem):
    wid = _wid()
    nbase = pl.multiple_of(wid * NPT, 8)
    pltpu.sync_copy(p_hbm, pbuf)
    pltpu.sync_copy(q_hbm, qbuf)
    pltpu.sync_copy(w_hbm, wbuf)
    pltpu.sync_copy(t_hbm, tbuf)
    pltpu.sync_copy(es_hbm, esb)

    zero = jnp.zeros((L,), _f32)

    def zero_body(i, c):
        for u in range(U):
            acc[i, pl.ds(u * L, L)] = zero
        return c

    lax.fori_loop(0, NPT, zero_body, 0)

    def zs_body(i, c):
        svec[pl.ds(i * L, L)] = zero
        return c

    lax.fori_loop(0, (NPT + L) // L, zs_body, 0)

    esv = esb[pl.ds(wid, L)]
    st = esv[0]
    en = esv[1]
    abase = pl.multiple_of(st & (-8), 8)
    nb = (en - abase + (B - 1)) // B

    def block_body(b, c):
        gbase = pl.multiple_of(abase + b * B, 8)
        pltpu.sync_copy(src_hbm.at[pl.ds(gbase, B)], sidx)
        pltpu.sync_copy(dst_hbm.at[pl.ds(gbase, B)], didx)
        pltpu.async_copy(h_hbm.at[sidx], rows, sem).wait()
        for g in range(B // L):
            sl = pl.ds(g * L, L)
            si = sidx[sl]
            di = didx[sl]
            pv = plsc.load_gather(pbuf, [si])
            qv = plsc.load_gather(qbuf, [di])
            e = pv + qv
            e = jnp.where(e > 0, e, NEG_SLOPE * e)
            exb[sl] = jnp.exp(e)
        lo = st - gbase
        hi = jnp.minimum(en - gbase, B)

        def grp_body(g, c2):
            off = g * L
            lane = off + _iota()
            valid = (lane >= lo) & (lane < hi)
            exv = jnp.where(valid, exb[pl.ds(off, L)], 0.0)
            dv = didx[pl.ds(off, L)] - nbase
            dv = jnp.minimum(jnp.maximum(dv, 0), NPT - 1)
            for j in range(L):
                exi = exv[j]
                dloc = dv[j]
                i = off + j
                plsc.addupdate(
                    svec.at[pl.ds(dloc, L)],
                    jnp.where(_iota() == 0, exi, 0.0),
                )
                for u in range(U):
                    sl = pl.ds(u * L, L)
                    plsc.addupdate(acc.at[dloc, sl], rows[i, sl] * exi)
            return c2

        lax.fori_loop(0, B // L, grp_body, 0)
        return c

    lax.fori_loop(0, nb, block_body, 0)

    def rs_body(i, c):
        sl = pl.ds(i * L, L)
        sv = svec[sl]
        rsb[sl] = jnp.where(sv > 0, 1.0 / sv, 0.0)
        return c

    lax.fori_loop(0, (NPT + L) // L, rs_body, 0)

    tk = tbuf[pl.ds(0, L)][1]

    for cc in range(NPT // 64):
        rbase = cc * 64
        pltpu.sync_copy(th_hbm.at[pl.ds(nbase + rbase, 64), :], thbuf)

        def fin_body(g, c, rbase=rbase):
            pvec = jnp.zeros((L,), _f32)
            qvec = jnp.zeros((L,), _f32)
            rsv = rsb[pl.ds(rbase + g * L, L)]
            for j in range(L):
                i = g * L + j
                row = rbase + i
                rs = rsv[j]
                dp = jnp.zeros((L,), _f32)
                dq = jnp.zeros((L,), _f32)
                for u in range(U):
                    sl = pl.ds(u * L, L)
                    hv = acc[row, sl] * rs
                    acc[row, sl] = hv
                    thbuf[i, sl] = thbuf[i, sl] + hv * tk
                    dp = dp + hv * wbuf[sl]
                    dq = dq + hv * wbuf[pl.ds(D + u * L, L)]
                ej = (_iota() == j)
                pvec = jnp.where(ej, _hsum(dp), pvec)
                qvec = jnp.where(ej, _hsum(dq), qvec)
            pb2[pl.ds(rbase + g * L, L)] = pvec
            qb2[pl.ds(rbase + g * L, L)] = qvec
            return c

        lax.fori_loop(0, 64 // L, fin_body, 0)
        pltpu.sync_copy(thbuf, th_out.at[pl.ds(nbase + rbase, 64), :])
        pltpu.sync_copy(acc.at[pl.ds(rbase, 64), :],
                        h_out.at[pl.ds(nbase + rbase, 64), :])
    pltpu.sync_copy(pb2, p_out.at[pl.ds(nbase, NPT)])
    pltpu.sync_copy(qb2, q_out.at[pl.ds(nbase, NPT)])


def kernel(in_feat, edge_index, temp, w_attn):
    src = edge_index[0].astype(_i32)
    dst = edge_index[1].astype(_i32)
    order = jnp.argsort(dst)
    src_s = jnp.take(src, order)
    dst_s = jnp.take(dst, order)
    estart = jnp.searchsorted(
        dst_s, jnp.arange(0, NPAD + 1, NPT, dtype=_i32)
    ).astype(_i32)
    estart = jnp.concatenate([estart, jnp.zeros((48 - estart.shape[0],), _i32)])
    src_p = jnp.concatenate([src_s, jnp.zeros((EPAD - E,), _i32)])
    dst_p = jnp.concatenate([dst_s, jnp.zeros((EPAD - E,), _i32)])
    h0 = jnp.zeros((NPAD, D), _f32).at[:N].set(in_feat)
    tpad = jnp.zeros((16,), _f32).at[: K + 1].set(temp)

    p, q, th = _init_kernel(h0, w_attn, tpad)
    h = h0
    for k in range(K):
        tk = jnp.zeros((16,), _f32).at[1].set(temp[k + 1])
        h, th, p, q = _step_kernel(h, th, p, q, src_p, dst_p, estart,
                                   w_attn, tk)
    return th[:N]


# speedup vs baseline: 10.2371x; 1.2289x over previous
import functools

import jax
import jax.numpy as jnp
from jax import lax
from jax.experimental import pallas as pl
from jax.experimental.pallas import tpu as pltpu
from jax.experimental.pallas import tpu_sc as plsc

N = 10000
D = 128
E = 320000
K = 10
NEG_SLOPE = 0.01

NC, NS, L = 2, 16, 16
T = NC * NS
NPT = 320
NPAD = NPT * T
B = 128
EPAD = E + 2 * B
U = D // L

_mesh = plsc.VectorSubcoreMesh(
    core_axis_name="c", subcore_axis_name="s", num_cores=NC, num_subcores=NS
)

_f32 = jnp.float32
_i32 = jnp.int32


def _wid():
    return lax.axis_index("c") * NS + lax.axis_index("s")


def _iota():
    return lax.iota(_i32, L)


_GDN = lax.GatherDimensionNumbers(
    offset_dims=(), collapsed_slice_dims=(0,), start_index_map=(0,)
)


def _rot(v, sh):
    idx = lax.iota(_i32, L)
    idx = (idx + sh) & (L - 1)
    return lax.gather(v, idx[:, None], _GDN, slice_sizes=(1,),
                      mode=lax.GatherScatterMode.PROMISE_IN_BOUNDS)


def _hsum(v):
    for sh in (8, 4, 2, 1):
        v = v + _rot(v, sh)
    return v


@functools.partial(
    pl.kernel,
    out_type=(
        jax.ShapeDtypeStruct((NPAD,), _f32),
        jax.ShapeDtypeStruct((NPAD,), _f32),
        jax.ShapeDtypeStruct((NPAD, D), _f32),
    ),
    mesh=_mesh,
    compiler_params=pltpu.CompilerParams(needs_layout_passes=False),
    scratch_types=[
        pltpu.VMEM((64, D), _f32),
        pltpu.VMEM((NPT,), _f32),
        pltpu.VMEM((NPT,), _f32),
        pltpu.VMEM((2 * D,), _f32),
        pltpu.VMEM((16,), _f32),
    ],
)
def _init_kernel(h_hbm, w_hbm, t_hbm, p_hbm, q_hbm, th_hbm, rbuf, pb, qb, wbuf, tbuf):
    base = pl.multiple_of(_wid() * NPT, 8)
    pltpu.sync_copy(w_hbm, wbuf)
    pltpu.sync_copy(t_hbm, tbuf)
    t0 = tbuf[pl.ds(0, L)][0]
    for cc in range(NPT // 64):
        rbase = cc * 64
        pltpu.sync_copy(h_hbm.at[pl.ds(base + rbase, 64), :], rbuf)

        def grp_body(g, c, rbase=rbase):
            pvec = jnp.zeros((L,), _f32)
            qvec = jnp.zeros((L,), _f32)
            for j in range(L):
                i = g * L + j
                dp = jnp.zeros((L,), _f32)
                dq = jnp.zeros((L,), _f32)
                for u in range(U):
                    sl = pl.ds(u * L, L)
                    hv = rbuf[i, sl]
                    dp = dp + hv * wbuf[sl]
                    dq = dq + hv * wbuf[pl.ds(D + u * L, L)]
                    rbuf[i, sl] = hv * t0
                ej = (_iota() == j)
                pvec = jnp.where(ej, _hsum(dp), pvec)
                qvec = jnp.where(ej, _hsum(dq), qvec)
            pb[pl.ds(rbase + g * L, L)] = pvec
            qb[pl.ds(rbase + g * L, L)] = qvec
            return c

        lax.fori_loop(0, 64 // L, grp_body, 0)
        pltpu.sync_copy(rbuf, th_hbm.at[pl.ds(base + rbase, 64), :])
    pltpu.sync_copy(pb, p_hbm.at[pl.ds(base, NPT)])
    pltpu.sync_copy(qb, q_hbm.at[pl.ds(base, NPT)])


@functools.partial(
    pl.kernel,
    out_type=(
        jax.ShapeDtypeStruct((NPAD, D), _f32),
        jax.ShapeDtypeStruct((NPAD, D), _f32),
        jax.ShapeDtypeStruct((NPAD,), _f32),
        jax.ShapeDtypeStruct((NPAD,), _f32),
    ),
    mesh=_mesh,
    compiler_params=pltpu.CompilerParams(needs_layout_passes=False),
    scratch_types=[
        pltpu.VMEM((NPAD,), _f32),
        pltpu.VMEM((NPAD,), _f32),
        pltpu.VMEM((NPT, D), _f32),
        pltpu.VMEM((NPT + L,), _f32),
        pltpu.VMEM((NPT + L,), _f32),
        pltpu.VMEM((B, D), _f32),
        pltpu.VMEM((B, D), _f32),
        pltpu.VMEM((B,), _i32),
        pltpu.VMEM((B,), _i32),
        pltpu.VMEM((B,), _f32),
        pltpu.VMEM((B,), _i32),
        pltpu.VMEM((B,), _i32),
        pltpu.VMEM((B,), _f32),
        pltpu.VMEM((64, D), _f32),
        pltpu.VMEM((2 * D,), _f32),
        pltpu.VMEM((16,), _f32),
        pltpu.VMEM((48,), _i32),
        pltpu.VMEM((NPT,), _f32),
        pltpu.VMEM((NPT,), _f32),
        pltpu.SemaphoreType.DMA,
        pltpu.SemaphoreType.DMA,
        pltpu.SemaphoreType.DMA,
        pltpu.SemaphoreType.DMA,
    ],
)
def _step_kernel(h_hbm, th_hbm, p_hbm, q_hbm, src_hbm, dst_hbm, es_hbm, w_hbm,
                 t_hbm, h_out, th_out, p_out, q_out,
                 pbuf, qbuf, acc, svec, rsb, rows, rows1, sidx, didx, exb,
                 sidx1, didx1, exb1, thbuf, wbuf, tbuf, esb, pb2, qb2,
                 isem0, isem1, gsem0, gsem1):
    wid = _wid()
    nbase = pl.multiple_of(wid * NPT, 8)
    pltpu.sync_copy(p_hbm, pbuf)
    pltpu.sync_copy(q_hbm, qbuf)
    pltpu.sync_copy(w_hbm, wbuf)
    pltpu.sync_copy(t_hbm, tbuf)
    pltpu.sync_copy(es_hbm, esb)

    zero = jnp.zeros((L,), _f32)

    def zero_body(i, c):
        for u in range(U):
            acc[i, pl.ds(u * L, L)] = zero
        return c

    lax.fori_loop(0, NPT, zero_body, 0)

    def zs_body(i, c):
        svec[pl.ds(i * L, L)] = zero
        return c

    lax.fori_loop(0, (NPT + L) // L, zs_body, 0)

    esv = esb[pl.ds(wid, L)]
    st = esv[0]
    en = esv[1]
    abase = pl.multiple_of(st & (-8), 8)
    nb = (en - abase + (B - 1)) // B
    nb2 = (nb + 1) // 2

    def _start_idx(b, sbuf, dbuf, sem):
        gb = pl.multiple_of(abase + b * B, 8)
        pltpu.async_copy(src_hbm.at[pl.ds(gb, B)], sbuf, sem)
        pltpu.async_copy(dst_hbm.at[pl.ds(gb, B)], dbuf, sem)

    def _wait_idx(b, sbuf, dbuf, sem):
        gb = pl.multiple_of(abase + b * B, 8)
        pltpu.make_async_copy(src_hbm.at[pl.ds(gb, B)], sbuf, sem).wait()
        pltpu.make_async_copy(dst_hbm.at[pl.ds(gb, B)], dbuf, sem).wait()

    def _ex_phase(sb, db, eb):
        for g in range(B // L):
            sl = pl.ds(g * L, L)
            pv = plsc.load_gather(pbuf, [sb[sl]])
            qv = plsc.load_gather(qbuf, [db[sl]])
            e = pv + qv
            e = jnp.where(e > 0, e, NEG_SLOPE * e)
            eb[sl] = jnp.exp(e)

    def _edge_phase(b, db, eb, rw):
        gbase = abase + b * B
        lo = st - gbase
        hi = jnp.minimum(en - gbase, B)

        def grp_body(g, c2):
            off = g * L
            lane = off + _iota()
            valid = (lane >= lo) & (lane < hi)
            exv = jnp.where(valid, eb[pl.ds(off, L)], 0.0)
            dv = db[pl.ds(off, L)] - nbase
            dv = jnp.minimum(jnp.maximum(dv, 0), NPT - 1)
            for j in range(L):
                exi = exv[j]
                dloc = dv[j]
                i = off + j
                plsc.addupdate(
                    svec.at[pl.ds(dloc, L)],
                    jnp.where(_iota() == 0, exi, 0.0),
                )
                for u in range(U):
                    sl = pl.ds(u * L, L)
                    plsc.addupdate(acc.at[dloc, sl], rw[i, sl] * exi)
            return c2

        lax.fori_loop(0, B // L, grp_body, 0)

    _start_idx(0, sidx, didx, isem0)

    def pair_body(m, c):
        b0 = m * 2
        b1 = b0 + 1
        _wait_idx(b0, sidx, didx, isem0)
        g0 = pltpu.async_copy(h_hbm.at[sidx], rows, gsem0)
        _start_idx(b1, sidx1, didx1, isem1)
        _ex_phase(sidx, didx, exb)
        _wait_idx(b1, sidx1, didx1, isem1)
        g1 = pltpu.async_copy(h_hbm.at[sidx1], rows1, gsem1)
        _ex_phase(sidx1, didx1, exb1)
        g0.wait()
        _edge_phase(b0, didx, exb, rows)
        _start_idx(b0 + 2, sidx, didx, isem0)
        g1.wait()
        _edge_phase(b1, didx1, exb1, rows1)
        return c

    lax.fori_loop(0, nb2, pair_body, 0)
    _wait_idx(nb2 * 2, sidx, didx, isem0)

    def rs_body(i, c):
        sl = pl.ds(i * L, L)
        sv = svec[sl]
        rsb[sl] = jnp.where(sv > 0, 1.0 / sv, 0.0)
        return c

    lax.fori_loop(0, (NPT + L) // L, rs_body, 0)

    tk = tbuf[pl.ds(0, L)][1]

    for cc in range(NPT // 64):
        rbase = cc * 64
        pltpu.sync_copy(th_hbm.at[pl.ds(nbase + rbase, 64), :], thbuf)

        def fin_body(g, c, rbase=rbase):
            pvec = jnp.zeros((L,), _f32)
            qvec = jnp.zeros((L,), _f32)
            rsv = rsb[pl.ds(rbase + g * L, L)]
            for j in range(L):
                i = g * L + j
                row = rbase + i
                rs = rsv[j]
                dp = jnp.zeros((L,), _f32)
                dq = jnp.zeros((L,), _f32)
                for u in range(U):
                    sl = pl.ds(u * L, L)
                    hv = acc[row, sl] * rs
                    acc[row, sl] = hv
                    thbuf[i, sl] = thbuf[i, sl] + hv * tk
                    dp = dp + hv * wbuf[sl]
                    dq = dq + hv * wbuf[pl.ds(D + u * L, L)]
                ej = (_iota() == j)
                pvec = jnp.where(ej, _hsum(dp), pvec)
                qvec = jnp.where(ej, _hsum(dq), qvec)
            pb2[pl.ds(rbase + g * L, L)] = pvec
            qb2[pl.ds(rbase + g * L, L)] = qvec
            return c

        lax.fori_loop(0, 64 // L, fin_body, 0)
        pltpu.sync_copy(thbuf, th_out.at[pl.ds(nbase + rbase, 64), :])
        pltpu.sync_copy(acc.at[pl.ds(rbase, 64), :],
                        h_out.at[pl.ds(nbase + rbase, 64), :])
    pltpu.sync_copy(pb2, p_out.at[pl.ds(nbase, NPT)])
    pltpu.sync_copy(qb2, q_out.at[pl.ds(nbase, NPT)])


def kernel(in_feat, edge_index, temp, w_attn):
    src = edge_index[0].astype(_i32)
    dst = edge_index[1].astype(_i32)
    order = jnp.argsort(dst)
    src_s = jnp.take(src, order)
    dst_s = jnp.take(dst, order)
    estart = jnp.searchsorted(
        dst_s, jnp.arange(0, NPAD + 1, NPT, dtype=_i32)
    ).astype(_i32)
    estart = jnp.concatenate([estart, jnp.zeros((48 - estart.shape[0],), _i32)])
    src_p = jnp.concatenate([src_s, jnp.zeros((EPAD - E,), _i32)])
    dst_p = jnp.concatenate([dst_s, jnp.zeros((EPAD - E,), _i32)])
    h0 = jnp.zeros((NPAD, D), _f32).at[:N].set(in_feat)
    tpad = jnp.zeros((16,), _f32).at[: K + 1].set(temp)

    p, q, th = _init_kernel(h0, w_attn, tpad)
    h = h0
    for k in range(K):
        tk = jnp.zeros((16,), _f32).at[1].set(temp[k + 1])
        h, th, p, q = _step_kernel(h, th, p, q, src_p, dst_p, estart,
                                   w_attn, tk)
    return th[:N]


# speedup vs baseline: 21.9115x; 2.1404x over previous
import functools

import jax
import jax.numpy as jnp
from jax import lax
from jax.experimental import pallas as pl
from jax.experimental.pallas import tpu as pltpu
from jax.experimental.pallas import tpu_sc as plsc

N = 10000
D = 128
E = 320000
K = 10
NEG_SLOPE = 0.01

NC, NS, L = 2, 16, 16
T = NC * NS
NPT = 320
NPAD = NPT * T
B = 128
EPAD = E + 2 * B
U = D // L

_mesh = plsc.VectorSubcoreMesh(
    core_axis_name="c", subcore_axis_name="s", num_cores=NC, num_subcores=NS
)

_f32 = jnp.float32
_i32 = jnp.int32


def _wid():
    return lax.axis_index("c") * NS + lax.axis_index("s")


def _iota():
    return lax.iota(_i32, L)


_GDN = lax.GatherDimensionNumbers(
    offset_dims=(), collapsed_slice_dims=(0,), start_index_map=(0,)
)


def _rot(v, sh):
    idx = lax.iota(_i32, L)
    idx = (idx + sh) & (L - 1)
    return lax.gather(v, idx[:, None], _GDN, slice_sizes=(1,),
                      mode=lax.GatherScatterMode.PROMISE_IN_BOUNDS)


def _hsum(v):
    for sh in (8, 4, 2, 1):
        v = v + _rot(v, sh)
    return v


@functools.partial(
    pl.kernel,
    out_type=(
        jax.ShapeDtypeStruct((NPAD,), _f32),
        jax.ShapeDtypeStruct((NPAD,), _f32),
        jax.ShapeDtypeStruct((NPAD, D), _f32),
    ),
    mesh=_mesh,
    compiler_params=pltpu.CompilerParams(needs_layout_passes=False),
    scratch_types=[
        pltpu.VMEM((64, D), _f32),
        pltpu.VMEM((NPT,), _f32),
        pltpu.VMEM((NPT,), _f32),
        pltpu.VMEM((2 * D,), _f32),
        pltpu.VMEM((16,), _f32),
    ],
)
def _init_kernel(h_hbm, w_hbm, t_hbm, p_hbm, q_hbm, th_hbm, rbuf, pb, qb, wbuf, tbuf):
    base = pl.multiple_of(_wid() * NPT, 8)
    pltpu.sync_copy(w_hbm, wbuf)
    pltpu.sync_copy(t_hbm, tbuf)
    t0 = tbuf[pl.ds(0, L)][0]
    for cc in range(NPT // 64):
        rbase = cc * 64
        pltpu.sync_copy(h_hbm.at[pl.ds(base + rbase, 64), :], rbuf)

        def grp_body(g, c, rbase=rbase):
            pvec = jnp.zeros((L,), _f32)
            qvec = jnp.zeros((L,), _f32)
            for j in range(L):
                i = g * L + j
                hvv = [rbuf[i, pl.ds(u * L, L)] for u in range(U)]
                dp = jnp.zeros((L,), _f32)
                dq = jnp.zeros((L,), _f32)
                for u in range(U):
                    sl = pl.ds(u * L, L)
                    hv = hvv[u]
                    dp = dp + hv * wbuf[sl]
                    dq = dq + hv * wbuf[pl.ds(D + u * L, L)]
                    rbuf[i, sl] = hv * t0
                ej = (_iota() == j)
                pvec = jnp.where(ej, _hsum(dp), pvec)
                qvec = jnp.where(ej, _hsum(dq), qvec)
            pb[pl.ds(rbase + g * L, L)] = pvec
            qb[pl.ds(rbase + g * L, L)] = qvec
            return c

        lax.fori_loop(0, 64 // L, grp_body, 0)
        pltpu.sync_copy(rbuf, th_hbm.at[pl.ds(base + rbase, 64), :])
    pltpu.sync_copy(pb, p_hbm.at[pl.ds(base, NPT)])
    pltpu.sync_copy(qb, q_hbm.at[pl.ds(base, NPT)])


@functools.partial(
    pl.kernel,
    out_type=(
        jax.ShapeDtypeStruct((NPAD, D), _f32),
        jax.ShapeDtypeStruct((NPAD, D), _f32),
        jax.ShapeDtypeStruct((NPAD,), _f32),
        jax.ShapeDtypeStruct((NPAD,), _f32),
    ),
    mesh=_mesh,
    compiler_params=pltpu.CompilerParams(needs_layout_passes=False),
    scratch_types=[
        pltpu.VMEM((NPAD,), _f32),
        pltpu.VMEM((NPAD,), _f32),
        pltpu.VMEM((NPT, D), _f32),
        pltpu.VMEM((NPT + L,), _f32),
        pltpu.VMEM((NPT + L,), _f32),
        pltpu.VMEM((B, D), _f32),
        pltpu.VMEM((B, D), _f32),
        pltpu.VMEM((B,), _i32),
        pltpu.VMEM((B,), _i32),
        pltpu.VMEM((B,), _f32),
        pltpu.VMEM((B,), _i32),
        pltpu.VMEM((B,), _i32),
        pltpu.VMEM((B,), _f32),
        pltpu.VMEM((64, D), _f32),
        pltpu.VMEM((2 * D,), _f32),
        pltpu.VMEM((16,), _f32),
        pltpu.VMEM((48,), _i32),
        pltpu.VMEM((NPT,), _f32),
        pltpu.VMEM((NPT,), _f32),
        pltpu.SemaphoreType.DMA,
        pltpu.SemaphoreType.DMA,
        pltpu.SemaphoreType.DMA,
        pltpu.SemaphoreType.DMA,
    ],
)
def _step_kernel(h_hbm, th_hbm, p_hbm, q_hbm, src_hbm, dst_hbm, es_hbm, w_hbm,
                 t_hbm, h_out, th_out, p_out, q_out,
                 pbuf, qbuf, acc, svec, rsb, rows, rows1, sidx, didx, exb,
                 sidx1, didx1, exb1, thbuf, wbuf, tbuf, esb, pb2, qb2,
                 isem0, isem1, gsem0, gsem1):
    wid = _wid()
    nbase = pl.multiple_of(wid * NPT, 8)
    pltpu.sync_copy(p_hbm, pbuf)
    pltpu.sync_copy(q_hbm, qbuf)
    pltpu.sync_copy(w_hbm, wbuf)
    pltpu.sync_copy(t_hbm, tbuf)
    pltpu.sync_copy(es_hbm, esb)

    zero = jnp.zeros((L,), _f32)

    def zero_body(i, c):
        for u in range(U):
            acc[i, pl.ds(u * L, L)] = zero
        return c

    lax.fori_loop(0, NPT, zero_body, 0)

    def zs_body(i, c):
        svec[pl.ds(i * L, L)] = zero
        return c

    lax.fori_loop(0, (NPT + L) // L, zs_body, 0)

    esv = esb[pl.ds(wid, L)]
    st = esv[0]
    en = esv[1]
    abase = pl.multiple_of(st & (-8), 8)
    nb = (en - abase + (B - 1)) // B
    nb2 = (nb + 1) // 2

    def _start_idx(b, sbuf, dbuf, sem):
        gb = pl.multiple_of(abase + b * B, 8)
        pltpu.async_copy(src_hbm.at[pl.ds(gb, B)], sbuf, sem)
        pltpu.async_copy(dst_hbm.at[pl.ds(gb, B)], dbuf, sem)

    def _wait_idx(b, sbuf, dbuf, sem):
        gb = pl.multiple_of(abase + b * B, 8)
        pltpu.make_async_copy(src_hbm.at[pl.ds(gb, B)], sbuf, sem).wait()
        pltpu.make_async_copy(dst_hbm.at[pl.ds(gb, B)], dbuf, sem).wait()

    def _ex_phase(sb, db, eb):
        for g in range(B // L):
            sl = pl.ds(g * L, L)
            pv = plsc.load_gather(pbuf, [sb[sl]])
            qv = plsc.load_gather(qbuf, [db[sl]])
            e = pv + qv
            e = jnp.where(e > 0, e, NEG_SLOPE * e)
            eb[sl] = jnp.exp(e)

    def _edge_phase(b, db, eb, rw):
        gbase = abase + b * B
        lo = st - gbase
        hi = jnp.minimum(en - gbase, B)

        one0 = (_iota() == 0)

        def grp_body(g, c2):
            off = g * L
            lane = off + _iota()
            valid = (lane >= lo) & (lane < hi)
            exv = jnp.where(valid, eb[pl.ds(off, L)], 0.0)
            dv = db[pl.ds(off, L)] - nbase
            dv = jnp.minimum(jnp.maximum(dv, 0), NPT - 1)
            for j0 in range(0, L, 2):
                ja, jb = j0, j0 + 1
                ia, ib = off + ja, off + jb
                da, dbi = dv[ja], dv[jb]
                ea, ebv = exv[ja], exv[jb]
                rva = [rw[ia, pl.ds(u * L, L)] for u in range(U)]
                rvb = [rw[ib, pl.ds(u * L, L)] for u in range(U)]
                plsc.addupdate(svec.at[pl.ds(da, L)],
                               jnp.where(one0, ea, 0.0))
                for u in range(U):
                    plsc.addupdate(acc.at[da, pl.ds(u * L, L)], rva[u] * ea)
                plsc.addupdate(svec.at[pl.ds(dbi, L)],
                               jnp.where(one0, ebv, 0.0))
                for u in range(U):
                    plsc.addupdate(acc.at[dbi, pl.ds(u * L, L)], rvb[u] * ebv)
            return c2

        lax.fori_loop(0, B // L, grp_body, 0)

    _start_idx(0, sidx, didx, isem0)

    def pair_body(m, c):
        b0 = m * 2
        b1 = b0 + 1
        _wait_idx(b0, sidx, didx, isem0)
        g0 = pltpu.async_copy(h_hbm.at[sidx], rows, gsem0)
        _start_idx(b1, sidx1, didx1, isem1)
        _ex_phase(sidx, didx, exb)
        _wait_idx(b1, sidx1, didx1, isem1)
        g1 = pltpu.async_copy(h_hbm.at[sidx1], rows1, gsem1)
        _ex_phase(sidx1, didx1, exb1)
        g0.wait()
        _edge_phase(b0, didx, exb, rows)
        _start_idx(b0 + 2, sidx, didx, isem0)
        g1.wait()
        _edge_phase(b1, didx1, exb1, rows1)
        return c

    lax.fori_loop(0, nb2, pair_body, 0)
    _wait_idx(nb2 * 2, sidx, didx, isem0)

    def rs_body(i, c):
        sl = pl.ds(i * L, L)
        sv = svec[sl]
        rsb[sl] = jnp.where(sv > 0, 1.0 / sv, 0.0)
        return c

    lax.fori_loop(0, (NPT + L) // L, rs_body, 0)

    tk = tbuf[pl.ds(0, L)][1]

    for cc in range(NPT // 64):
        rbase = cc * 64
        pltpu.sync_copy(th_hbm.at[pl.ds(nbase + rbase, 64), :], thbuf)

        def fin_body(g, c, rbase=rbase):
            pvec = jnp.zeros((L,), _f32)
            qvec = jnp.zeros((L,), _f32)
            rsv = rsb[pl.ds(rbase + g * L, L)]
            for j in range(L):
                i = g * L + j
                row = rbase + i
                rs = rsv[j]
                av = [acc[row, pl.ds(u * L, L)] for u in range(U)]
                tv = [thbuf[i, pl.ds(u * L, L)] for u in range(U)]
                hvs = [av[u] * rs for u in range(U)]
                dp = jnp.zeros((L,), _f32)
                dq = jnp.zeros((L,), _f32)
                for u in range(U):
                    sl = pl.ds(u * L, L)
                    hv = hvs[u]
                    acc[row, sl] = hv
                    thbuf[i, sl] = tv[u] + hv * tk
                    dp = dp + hv * wbuf[sl]
                    dq = dq + hv * wbuf[pl.ds(D + u * L, L)]
                ej = (_iota() == j)
                pvec = jnp.where(ej, _hsum(dp), pvec)
                qvec = jnp.where(ej, _hsum(dq), qvec)
            pb2[pl.ds(rbase + g * L, L)] = pvec
            qb2[pl.ds(rbase + g * L, L)] = qvec
            return c

        lax.fori_loop(0, 64 // L, fin_body, 0)
        pltpu.sync_copy(thbuf, th_out.at[pl.ds(nbase + rbase, 64), :])
        pltpu.sync_copy(acc.at[pl.ds(rbase, 64), :],
                        h_out.at[pl.ds(nbase + rbase, 64), :])
    pltpu.sync_copy(pb2, p_out.at[pl.ds(nbase, NPT)])
    pltpu.sync_copy(qb2, q_out.at[pl.ds(nbase, NPT)])


def kernel(in_feat, edge_index, temp, w_attn):
    src = edge_index[0].astype(_i32)
    dst = edge_index[1].astype(_i32)
    order = jnp.argsort(dst)
    src_s = jnp.take(src, order)
    dst_s = jnp.take(dst, order)
    estart = jnp.searchsorted(
        dst_s, jnp.arange(0, NPAD + 1, NPT, dtype=_i32)
    ).astype(_i32)
    estart = jnp.concatenate([estart, jnp.zeros((48 - estart.shape[0],), _i32)])
    src_p = jnp.concatenate([src_s, jnp.zeros((EPAD - E,), _i32)])
    dst_p = jnp.concatenate([dst_s, jnp.zeros((EPAD - E,), _i32)])
    h0 = jnp.zeros((NPAD, D), _f32).at[:N].set(in_feat)
    tpad = jnp.zeros((16,), _f32).at[: K + 1].set(temp)

    p, q, th = _init_kernel(h0, w_attn, tpad)
    h = h0
    for k in range(K):
        tk = jnp.zeros((16,), _f32).at[1].set(temp[k + 1])
        h, th, p, q = _step_kernel(h, th, p, q, src_p, dst_p, estart,
                                   w_attn, tk)
    return th[:N]
